# baseline (speedup 1.0000x reference)
import jax
import jax.numpy as jnp
from jax.experimental import pallas as pl
from jax.experimental.pallas import tpu as pltpu

_NUM_C = 1024
_NUM_Q = 100000
_NUM_O = 4
_NCID = 8
_TOPK = 10
_NEG = -3.0e38


def _dott(a, b):
    return jax.lax.dot_general(a, b, (((1,), (1,)), ((), ())),
                               preferred_element_type=jnp.float32)


def _dotn(a, b):
    return jax.lax.dot_general(a, b, (((1,), (0,)), ((), ())),
                               preferred_element_type=jnp.float32)


def _mlp(x, w1, b1, w2, b2):
    h = jnp.maximum(_dott(x, w1) + b1, 0.0)
    return _dott(h, w2) + b2


def _ln(x, g, b):
    m = jnp.mean(x, axis=-1, keepdims=True)
    v = jnp.mean((x - m) ** 2, axis=-1, keepdims=True)
    return (x - m) / jnp.sqrt(v + 1e-5) * g + b


def _attn1(x_q, x_v, wv, bv, wo, bo, g, b):
    o = _dott(_dott(x_v, wv) + bv, wo) + bo
    return _ln(x_q + o, g, b)


def _body(rows_ref, cids_ref, scal_ref,
          qt_ref, ot_ref, ut_ref, mv_ref, lt_ref, mk_ref,
          ecw1_ref, ecb1_ref, ecw2_ref, ecb2_ref,
          eww1_ref, ewb1_ref, eww2_ref, ewb2_ref,
          euw1_ref, eub1_ref, euw2_ref, eub2_ref,
          rwv_ref, rbv_ref, rwo_ref, rbo_ref, rg_ref, rb_ref,
          qwv_ref, qbv_ref, qwo_ref, qbo_ref, qg_ref, qb_ref,
          swv_ref, sbv_ref, swo_ref, sbo_ref, sg_ref, sb_ref,
          fwm_ref,
          pjw_ref, pjb_ref,
          g1w_ref, g1s_ref, g1d_ref, g1b_ref,
          g2w_ref, g2s_ref, g2d_ref, g2b_ref,
          out_ref):
    f32 = jnp.float32
    qt = qt_ref[0]
    ot = ot_ref[0]
    ut = ut_ref[0]
    sc = scal_ref[0]
    ts = scal_ref[1]

    ot_c = _mlp(ot, ecw1_ref[...], ecb1_ref[...], ecw2_ref[...], ecb2_ref[...])
    ot_w = _mlp(ot, eww1_ref[...], ewb1_ref[...], eww2_ref[...], ewb2_ref[...])
    w_c = (sc == 1.0).astype(f32)
    w_w = (sc == 0.0).astype(f32)
    ot_p = w_c * ot_c + w_w * ot_w
    ut_p = _mlp(ut, euw1_ref[...], eub1_ref[...], euw2_ref[...], eub2_ref[...])
    d = ot_p - ut_p

    d_hat = _attn1(d, d, rwv_ref[...], rbv_ref[...], rwo_ref[...],
                   rbo_ref[...], rg_ref[...], rb_ref[...])
    qt_hat = _attn1(qt, qt, qwv_ref[...], qbv_ref[...], qwo_ref[...],
                    qbo_ref[...], qg_ref[...], qb_ref[...])
    h_cid = _attn1(qt_hat, d_hat, swv_ref[...], sbv_ref[...], swo_ref[...],
                   sbo_ref[...], sg_ref[...], sb_ref[...])

    iota_c = jax.lax.broadcasted_iota(jnp.int32, (_NUM_C, 1), 0)
    member = (iota_c == cids_ref[0]).astype(f32)
    for k in range(1, _NCID):
        member = jnp.maximum(member, (iota_c == cids_ref[k]).astype(f32))
    delta = member * (ts - lt_ref[...])
    time_feat = jnp.log1p(delta) * 0.5
    resp_upd = member * h_cid

    mv = mv_ref[...]
    ones_c = jnp.ones((_NUM_C, 1), f32)
    gin = jnp.concatenate([mv, time_feat, ones_c], axis=1)
    gate = _dott(gin, fwm_ref[...])
    gamma = jax.nn.sigmoid(gate)
    mv_new = gamma * mv + (1.0 - gamma) * resp_upd

    nrm = mv_new / jnp.maximum(
        jnp.sqrt(jnp.sum(mv_new * mv_new, axis=1, keepdims=True)), 1e-6)
    sim = _dott(nrm, nrm)
    ri = jax.lax.broadcasted_iota(jnp.int32, (_NUM_C, _NUM_C), 0)
    ci = jax.lax.broadcasted_iota(jnp.int32, (_NUM_C, _NUM_C), 1)
    adj = jnp.logical_or(sim > 0.05, ri == ci)

    def gat(x, w, a_s, a_d, b):
        h = _dott(x, w)
        sd = _dott(h, a_d)
        ss = _dott(a_s, h)
        e = sd + ss
        e = jnp.where(e >= 0.0, e, 0.2 * e)
        e = jnp.where(adj, e, _NEG)
        m = jnp.max(e, axis=1, keepdims=True)
        p = jnp.where(adj, jnp.exp(e - m), 0.0)
        a = p / jnp.sum(p, axis=1, keepdims=True)
        return _dotn(a, h) + b

    x1 = gat(mv_new, g1w_ref[...], g1s_ref[...], g1d_ref[...], g1b_ref[...])
    x1 = jnp.where(x1 > 0.0, x1, jnp.exp(jnp.minimum(x1, 0.0)) - 1.0)
    mv_gat = gat(x1, g2w_ref[...], g2s_ref[...], g2d_ref[...], g2b_ref[...])

    pq = _dott(qt_hat, pjw_ref[...]) + pjb_ref[...]
    simk = _dott(pq, mk_ref[...])
    pos = jax.lax.broadcasted_iota(jnp.int32, (1, _NUM_C), 1).astype(f32)

    def pick(_, carry):
        w, keep = carry
        m = jnp.max(w, axis=1, keepdims=True)
        ismax = (w == m).astype(f32)
        idx = jnp.min(jnp.where(ismax > 0.0, pos, f32(_NUM_C)),
                      axis=1, keepdims=True)
        sel = (pos == idx).astype(f32)
        return w + sel * _NEG, jnp.maximum(keep, sel)

    _, keep = jax.lax.fori_loop(
        0, _TOPK, pick, (simk, jnp.zeros((1, _NUM_C), f32)))
    masked = jnp.where(keep > 0.0, simk, _NEG)
    m = jnp.max(masked, axis=1, keepdims=True)
    p = keep * jnp.exp(simk - m)
    attn = p / jnp.sum(p, axis=1, keepdims=True)
    mastery = _dotn(attn, mv_gat)
    logit = jnp.sum(pq * mastery, axis=-1, keepdims=True)
    out_ref[...] = jax.nn.sigmoid(logit).astype(f32)


def kernel(student_id, q_idx, o_idx, u_idx, score, timestamp, concept_ids,
           mv, last_time, params):
    f32 = jnp.float32
    q = jnp.clip(q_idx[0], 0, _NUM_Q - 1).astype(jnp.int32)
    o = jnp.clip(o_idx[0], 0, _NUM_O - 1).astype(jnp.int32)
    u = jnp.clip(u_idx[0], 0, _NUM_O - 1).astype(jnp.int32)
    rows = jnp.stack([q, q * _NUM_O + o, q * _NUM_O + u])
    cids = concept_ids.astype(jnp.int32)

    qe = params["question_emb"].reshape(-1, 1, 64)
    re = params["response_emb"].reshape(-1, 1, 64)
    fw = params["forget_W"]
    scal = jnp.concatenate([score.astype(f32), timestamp.astype(f32)])

    def mlp_p(p):
        return [p["W1"], p["b1"].reshape(1, -1), p["W2"], p["b2"].reshape(1, -1)]

    def attn_p(p):
        E = 64
        return [p["Wi"][2 * E:], p["bi"][2 * E:].reshape(1, E), p["Wo"],
                p["bo"].reshape(1, E), p["g"].reshape(1, E), p["b"].reshape(1, E)]

    tensors = [
        qe, re, re,
        mv, last_time.reshape(_NUM_C, 1), params["memory_key"],
        *mlp_p(params["enc_correct"]), *mlp_p(params["enc_wrong"]),
        *mlp_p(params["enc_unchosen"]),
        *attn_p(params["attn_resp"]), *attn_p(params["attn_q"]),
        *attn_p(params["attn_s"]),
        jnp.concatenate([fw, params["forget_b"].reshape(1, 1)], axis=1),
        params["qproj_W"], params["qproj_b"].reshape(1, 64),
        params["gat1_W"], params["gat1_as"].reshape(1, 32),
        params["gat1_ad"].reshape(1, 32), params["gat1_b"].reshape(1, 32),
        params["gat2_W"], params["gat2_as"].reshape(1, 64),
        params["gat2_ad"].reshape(1, 64), params["gat2_b"].reshape(1, 64),
    ]

    def full_spec(t):
        shp = t.shape
        return pl.BlockSpec(shp, lambda i, r, c, _n=len(shp): (0,) * _n)

    in_specs = [
        pl.BlockSpec(memory_space=pltpu.SMEM),
        pl.BlockSpec((1, 1, 64), lambda i, r, c: (r[0], 0, 0)),
        pl.BlockSpec((1, 1, 64), lambda i, r, c: (r[1], 0, 0)),
        pl.BlockSpec((1, 1, 64), lambda i, r, c: (r[2], 0, 0)),
    ] + [full_spec(t) for t in tensors[3:]]

    grid_spec = pltpu.PrefetchScalarGridSpec(
        num_scalar_prefetch=2,
        grid=(1,),
        in_specs=in_specs,
        out_specs=pl.BlockSpec((1, 1), lambda i, r, c: (0, 0)),
    )
    out = pl.pallas_call(
        _body,
        grid_spec=grid_spec,
        out_shape=jax.ShapeDtypeStruct((1, 1), f32),
    )(rows, cids, scal, *tensors)
    return out.reshape(1)


# speedup vs baseline: 2.6170x; 2.6170x over previous
import jax
import jax.numpy as jnp
from jax.experimental import pallas as pl
from jax.experimental.pallas import tpu as pltpu

_NUM_C = 1024
_NUM_Q = 100000
_NUM_O = 4
_NCID = 8
_TOPK = 10
_NEG = -3.0e38


def _dott(a, b):
    return jax.lax.dot_general(a, b, (((1,), (1,)), ((), ())),
                               preferred_element_type=jnp.float32)


def _dotn(a, b):
    return jax.lax.dot_general(a, b, (((1,), (0,)), ((), ())),
                               preferred_element_type=jnp.float32)


def _mlp(x, w1, b1, w2, b2):
    h = jnp.maximum(_dott(x, w1) + b1, 0.0)
    return _dott(h, w2) + b2


def _ln(x, g, b):
    m = jnp.mean(x, axis=-1, keepdims=True)
    v = jnp.mean((x - m) ** 2, axis=-1, keepdims=True)
    return (x - m) / jnp.sqrt(v + 1e-5) * g + b


def _attn1(x_q, x_v, wv, bv, wo, bo, g, b):
    o = _dott(_dott(x_v, wv) + bv, wo) + bo
    return _ln(x_q + o, g, b)


def _body(rows_ref, cids_ref, scal_ref,
          qt_ref, ot_ref, ut_ref, mv_ref, lt_ref, mk_ref,
          ecw1_ref, ecb1_ref, ecw2_ref, ecb2_ref,
          eww1_ref, ewb1_ref, eww2_ref, ewb2_ref,
          euw1_ref, eub1_ref, euw2_ref, eub2_ref,
          rwv_ref, rbv_ref, rwo_ref, rbo_ref, rg_ref, rb_ref,
          qwv_ref, qbv_ref, qwo_ref, qbo_ref, qg_ref, qb_ref,
          swv_ref, sbv_ref, swo_ref, sbo_ref, sg_ref, sb_ref,
          fwm_ref,
          pjw_ref, pjb_ref,
          g1w_ref, g1s_ref, g1d_ref, g1b_ref,
          g2w_ref, g2s_ref, g2d_ref, g2b_ref,
          out_ref):
    f32 = jnp.float32
    qt = qt_ref[pl.ds(rows_ref[3], 1)]
    ot = ot_ref[pl.ds(rows_ref[4], 1)]
    ut = ut_ref[pl.ds(rows_ref[5], 1)]
    sc = scal_ref[0]
    ts = scal_ref[1]

    ot_c = _mlp(ot, ecw1_ref[...], ecb1_ref[...], ecw2_ref[...], ecb2_ref[...])
    ot_w = _mlp(ot, eww1_ref[...], ewb1_ref[...], eww2_ref[...], ewb2_ref[...])
    w_c = (sc == 1.0).astype(f32)
    w_w = (sc == 0.0).astype(f32)
    ot_p = w_c * ot_c + w_w * ot_w
    ut_p = _mlp(ut, euw1_ref[...], eub1_ref[...], euw2_ref[...], eub2_ref[...])
    d = ot_p - ut_p

    d_hat = _attn1(d, d, rwv_ref[...], rbv_ref[...], rwo_ref[...],
                   rbo_ref[...], rg_ref[...], rb_ref[...])
    qt_hat = _attn1(qt, qt, qwv_ref[...], qbv_ref[...], qwo_ref[...],
                    qbo_ref[...], qg_ref[...], qb_ref[...])
    h_cid = _attn1(qt_hat, d_hat, swv_ref[...], sbv_ref[...], swo_ref[...],
                   sbo_ref[...], sg_ref[...], sb_ref[...])

    iota_c = jax.lax.broadcasted_iota(jnp.int32, (_NUM_C, 1), 0)
    member = (iota_c == cids_ref[0]).astype(f32)
    for k in range(1, _NCID):
        member = jnp.maximum(member, (iota_c == cids_ref[k]).astype(f32))
    delta = member * (ts - lt_ref[...])
    time_feat = jnp.log1p(delta) * 0.5
    resp_upd = member * h_cid

    mv = mv_ref[...]
    ones_c = jnp.ones((_NUM_C, 1), f32)
    gin = jnp.concatenate([mv, time_feat, ones_c], axis=1)
    gate = _dott(gin, fwm_ref[...])
    gamma = jax.nn.sigmoid(gate)
    mv_new = gamma * mv + (1.0 - gamma) * resp_upd

    nrm = mv_new / jnp.maximum(
        jnp.sqrt(jnp.sum(mv_new * mv_new, axis=1, keepdims=True)), 1e-6)
    sim = _dott(nrm, nrm)
    ri = jax.lax.broadcasted_iota(jnp.int32, (_NUM_C, _NUM_C), 0)
    ci = jax.lax.broadcasted_iota(jnp.int32, (_NUM_C, _NUM_C), 1)
    adj = jnp.logical_or(sim > 0.05, ri == ci)

    def gat(x, w, a_s, a_d, b):
        h = _dott(x, w)
        sd = _dott(h, a_d)
        ss = _dott(a_s, h)
        e = sd + ss
        e = jnp.where(e >= 0.0, e, 0.2 * e)
        e = jnp.where(adj, e, _NEG)
        m = jnp.max(e, axis=1, keepdims=True)
        p = jnp.where(adj, jnp.exp(e - m), 0.0)
        a = p / jnp.sum(p, axis=1, keepdims=True)
        return _dotn(a, h) + b

    x1 = gat(mv_new, g1w_ref[...], g1s_ref[...], g1d_ref[...], g1b_ref[...])
    x1 = jnp.where(x1 > 0.0, x1, jnp.exp(jnp.minimum(x1, 0.0)) - 1.0)
    mv_gat = gat(x1, g2w_ref[...], g2s_ref[...], g2d_ref[...], g2b_ref[...])

    pq = _dott(qt_hat, pjw_ref[...]) + pjb_ref[...]
    simk = _dott(pq, mk_ref[...])
    pos = jax.lax.broadcasted_iota(jnp.int32, (1, _NUM_C), 1).astype(f32)

    def pick(_, carry):
        w, keep = carry
        m = jnp.max(w, axis=1, keepdims=True)
        ismax = (w == m).astype(f32)
        idx = jnp.min(jnp.where(ismax > 0.0, pos, f32(_NUM_C)),
                      axis=1, keepdims=True)
        sel = (pos == idx).astype(f32)
        return w + sel * _NEG, jnp.maximum(keep, sel)

    _, keep = jax.lax.fori_loop(
        0, _TOPK, pick, (simk, jnp.zeros((1, _NUM_C), f32)))
    masked = jnp.where(keep > 0.0, simk, _NEG)
    m = jnp.max(masked, axis=1, keepdims=True)
    p = keep * jnp.exp(simk - m)
    attn = p / jnp.sum(p, axis=1, keepdims=True)
    mastery = _dotn(attn, mv_gat)
    logit = jnp.sum(pq * mastery, axis=-1, keepdims=True)
    out_ref[...] = jax.nn.sigmoid(logit).astype(f32)


def kernel(student_id, q_idx, o_idx, u_idx, score, timestamp, concept_ids,
           mv, last_time, params):
    f32 = jnp.float32
    q = jnp.clip(q_idx[0], 0, _NUM_Q - 1).astype(jnp.int32)
    o = jnp.clip(o_idx[0], 0, _NUM_O - 1).astype(jnp.int32)
    u = jnp.clip(u_idx[0], 0, _NUM_O - 1).astype(jnp.int32)
    r3 = jnp.stack([q, q * _NUM_O + o, q * _NUM_O + u])
    rows = jnp.concatenate([r3 // 8, r3 % 8])
    cids = concept_ids.astype(jnp.int32)

    qe = params["question_emb"]
    re = params["response_emb"]
    fw = params["forget_W"]
    scal = jnp.concatenate([score.astype(f32), timestamp.astype(f32)])

    def mlp_p(p):
        return [p["W1"], p["b1"].reshape(1, -1), p["W2"], p["b2"].reshape(1, -1)]

    def attn_p(p):
        E = 64
        return [p["Wi"][2 * E:], p["bi"][2 * E:].reshape(1, E), p["Wo"],
                p["bo"].reshape(1, E), p["g"].reshape(1, E), p["b"].reshape(1, E)]

    tensors = [
        qe, re, re,
        mv, last_time.reshape(_NUM_C, 1), params["memory_key"],
        *mlp_p(params["enc_correct"]), *mlp_p(params["enc_wrong"]),
        *mlp_p(params["enc_unchosen"]),
        *attn_p(params["attn_resp"]), *attn_p(params["attn_q"]),
        *attn_p(params["attn_s"]),
        jnp.concatenate([fw, params["forget_b"].reshape(1, 1)], axis=1),
        params["qproj_W"], params["qproj_b"].reshape(1, 64),
        params["gat1_W"], params["gat1_as"].reshape(1, 32),
        params["gat1_ad"].reshape(1, 32), params["gat1_b"].reshape(1, 32),
        params["gat2_W"], params["gat2_as"].reshape(1, 64),
        params["gat2_ad"].reshape(1, 64), params["gat2_b"].reshape(1, 64),
    ]

    def full_spec(t):
        shp = t.shape
        return pl.BlockSpec(shp, lambda i, r, c, _n=len(shp): (0,) * _n)

    in_specs = [
        pl.BlockSpec(memory_space=pltpu.SMEM),
        pl.BlockSpec((8, 64), lambda i, r, c: (r[0], 0)),
        pl.BlockSpec((8, 64), lambda i, r, c: (r[1], 0)),
        pl.BlockSpec((8, 64), lambda i, r, c: (r[2], 0)),
    ] + [full_spec(t) for t in tensors[3:]]

    grid_spec = pltpu.PrefetchScalarGridSpec(
        num_scalar_prefetch=2,
        grid=(1,),
        in_specs=in_specs,
        out_specs=pl.BlockSpec((1, 1), lambda i, r, c: (0, 0)),
    )
    out = pl.pallas_call(
        _body,
        grid_spec=grid_spec,
        out_shape=jax.ShapeDtypeStruct((1, 1), f32),
    )(rows, cids, scal, *tensors)
    return out.reshape(1)
